# baseline (speedup 1.0000x reference)
import functools

import jax
import jax.numpy as jnp
from jax import lax
from jax.experimental import pallas as pl
from jax.experimental.pallas import tpu as pltpu
from jax.experimental.pallas import tpu_sc as plsc

B = 64
N = 1000000
TOPK = 5
L = 16
CHUNK = 20000
NCHUNK = N // CHUNK
VPC = CHUNK // L
UNROLL = 10
NC = 2
NS = 16
NW = NC * NS
ROWS_PER_TILE = B // NW

NEG = float("-inf")


def _insert(t, v):
    out = []
    r = v
    for k in range(TOPK):
        out.append(jnp.maximum(t[k], r))
        r = jnp.minimum(t[k], r)
    return out


def _merge5(a, b):
    c = []
    for k in range(1, TOPK + 1):
        cands = []
        for i in range(0, k + 1):
            j = k - i
            if i == 0:
                cands.append(b[j - 1])
            elif j == 0:
                cands.append(a[i - 1])
            else:
                cands.append(jnp.minimum(a[i - 1], b[j - 1]))
        m = cands[0]
        for x in cands[1:]:
            m = jnp.maximum(m, x)
        c.append(m)
    return c


def _scan_body(inp, tgt, out, buf0, buf1, tgtv, rot, res, sem0, sem1):
    wid = lax.axis_index("s") * NC + lax.axis_index("c")
    iota = lax.iota(jnp.int32, L)

    row0 = wid * ROWS_PER_TILE
    tblk = (row0 // L) * L
    pltpu.sync_copy(tgt.at[pl.ds(tblk, L)], tgtv.at[pl.ds(0, L)])
    pltpu.sync_copy(tgt.at[pl.ds(tblk, L)], tgtv.at[pl.ds(L, L)])

    results = []
    for r in range(ROWS_PER_TILE):
        row = row0 + r
        lane = row - (row // L) * L
        tgt_s = tgtv[pl.ds(row - tblk, L)][0]
        base = row * N

        pltpu.async_copy(inp.at[pl.ds(base, CHUNK)], buf0, sem0)
        pltpu.async_copy(inp.at[pl.ds(base + CHUNK, CHUNK)], buf1, sem1)

        def chunk_step(buf, c, carry):
            t1, t2, t3, t4, t5, gtv = carry

            def vec_group(i, tc):
                t = list(tc)
                for u in range(UNROLL):
                    v = buf[pl.ds((i * UNROLL + u) * L, L)]
                    t = _insert(t, v)
                return tuple(t)

            t1, t2, t3, t4, t5 = lax.fori_loop(
                0, VPC // UNROLL, vec_group, (t1, t2, t3, t4, t5))

            in_chunk = (tgt_s >= c * CHUNK) & (tgt_s < (c + 1) * CHUNK)
            off = jnp.clip(tgt_s - c * CHUNK, 0, CHUNK - 1)
            voff = (off // L) * L
            vlane = jnp.where(in_chunk, off - voff, L)
            v = buf[pl.ds(voff, L)]
            gtv = jnp.maximum(gtv, jnp.where(iota == vlane, v, NEG))
            return (t1, t2, t3, t4, t5, gtv)

        def outer(cc, carry):
            c0 = cc * 2
            pltpu.make_async_copy(inp.at[pl.ds(base, CHUNK)], buf0, sem0).wait()
            carry = chunk_step(buf0, c0, carry)

            @pl.when(c0 + 2 < NCHUNK)
            def _():
                pltpu.async_copy(
                    inp.at[pl.ds(base + (c0 + 2) * CHUNK, CHUNK)], buf0, sem0)

            pltpu.make_async_copy(inp.at[pl.ds(base, CHUNK)], buf1, sem1).wait()
            carry = chunk_step(buf1, c0 + 1, carry)

            @pl.when(c0 + 3 < NCHUNK)
            def _():
                pltpu.async_copy(
                    inp.at[pl.ds(base + (c0 + 3) * CHUNK, CHUNK)], buf1, sem1)

            return carry

        neg = jnp.full((L,), NEG, jnp.float32)
        carry = (neg, neg, neg, neg, neg, neg)
        carry = lax.fori_loop(0, NCHUNK // 2, outer, carry)
        t = list(carry[:TOPK])
        gtv = carry[TOPK]

        for sh in (8, 4, 2, 1):
            bs = []
            for k in range(TOPK):
                rot[k, pl.ds(0, L)] = t[k]
                rot[k, pl.ds(L, L)] = t[k]
            for k in range(TOPK):
                bs.append(rot[k, pl.ds(sh, L)])
            t = _merge5(t, bs)

        for sh in (8, 4, 2, 1):
            rot[0, pl.ds(0, L)] = gtv
            rot[0, pl.ds(L, L)] = gtv
            gtv = jnp.maximum(gtv, rot[0, pl.ds(sh, L)])

        results.append((t[TOPK - 1], gtv))

    resv = jnp.full((L,), 0.0, jnp.float32)
    resv = jnp.where(iota == 0, results[0][0], resv)
    resv = jnp.where(iota == 1, results[1][0], resv)
    resv = jnp.where(iota == 2, results[0][1], resv)
    resv = jnp.where(iota == 3, results[1][1], resv)
    res[...] = resv
    pltpu.sync_copy(res, out.at[wid])


_scan_kernel = functools.partial(
    pl.kernel,
    out_type=jax.ShapeDtypeStruct((NW, L), jnp.float32),
    mesh=plsc.VectorSubcoreMesh(core_axis_name="c", subcore_axis_name="s"),
    scratch_types=[
        pltpu.VMEM((CHUNK,), jnp.float32),
        pltpu.VMEM((CHUNK,), jnp.float32),
        pltpu.VMEM((2 * L,), jnp.int32),
        pltpu.VMEM((TOPK, 2 * L), jnp.float32),
        pltpu.VMEM((L,), jnp.float32),
        pltpu.SemaphoreType.DMA,
        pltpu.SemaphoreType.DMA,
    ],
)(_scan_body)


def _acc_body(gt_ref, thr_ref, o_ref):
    cmp = (gt_ref[...] >= thr_ref[...]).astype(jnp.float32)
    o_ref[...] = (jnp.sum(cmp) * (1.0 / (B * B))).reshape(1, 1)


def kernel(input, target):
    inp_flat = input.reshape(-1)
    tgt = target.astype(jnp.int32)
    out32 = _scan_kernel(inp_flat, tgt)
    thr = out32[:, 0:2].reshape(1, B)
    gt = out32[:, 2:4].reshape(B, 1)
    acc = pl.pallas_call(
        _acc_body,
        out_shape=jax.ShapeDtypeStruct((1, 1), jnp.float32),
    )(gt, thr)
    return acc[0, 0]


# speedup vs baseline: 4.2646x; 4.2646x over previous
import functools

import jax
import jax.numpy as jnp
from jax import lax
from jax.experimental import pallas as pl
from jax.experimental.pallas import tpu as pltpu
from jax.experimental.pallas import tpu_sc as plsc

B = 64
N = 1000000
TOPK = 5
L = 16
NC = 2
NS = 16
NW = NC * NS
NSLAB = 4
TILES_PER_SLAB = 1953
SLAB_COLS = TILES_PER_SLAB * 128
CHUNK_COLS = 256
NCH = 976
TAIL_COL = NCH * CHUNK_COLS
PART_COL = NSLAB * SLAB_COLS

NEG = float("-inf")


def _insert(t, v):
    out = []
    r = v
    for k in range(TOPK):
        out.append(jnp.maximum(t[k], r))
        r = jnp.minimum(t[k], r)
    return tuple(out)


def _scan_body(inp, tgt, out, bufA, bufB, bufT, bufP, tgtv, loc, shared,
               sem0, sem1, sem2):
    iota = lax.iota(jnp.int32, L)
    negv = jnp.full((L,), NEG, jnp.float32)
    core = lax.axis_index("c")
    sid = lax.axis_index("s")
    wid = core * NS + sid
    tr = wid // NSLAB
    sl = wid - tr * NSLAB
    r0 = pl.multiple_of(tr * 8, 8)
    colbase = sl * SLAB_COLS

    off0 = pl.multiple_of(jnp.minimum(r0, B - L), 8)
    pltpu.sync_copy(tgt.at[pl.ds(off0, L)], tgtv.at[pl.ds(0, L)])
    pltpu.sync_copy(tgt.at[pl.ds(off0, L)], tgtv.at[pl.ds(L, L)])
    idxbase = r0 - off0
    tcols = [tgtv[pl.ds(idxbase + s, L)][0] for s in range(8)]

    for s in range(8):
        loc[pl.ds((40 + s) * L, L)] = negv

    def start(c, buf, sem):
        col = pl.multiple_of(colbase + c * CHUNK_COLS, 128)
        pltpu.async_copy(inp.at[pl.ds(r0, 8), pl.ds(col, CHUNK_COLS)],
                         buf, sem)

    def wait(buf, sem, width):
        pltpu.make_async_copy(inp.at[pl.ds(r0, 8), pl.ds(0, width)],
                              buf, sem).wait()

    def gt_probe(buf, width, col0, s):
        cs = tcols[s] - col0

        @pl.when((cs >= 0) & (cs < width))
        def _():
            acc = loc[pl.ds((40 + s) * L, L)]
            for g in range(width // L):
                v = buf[s, pl.ds(g * L, L)]
                acc = jnp.maximum(acc, jnp.where(iota == cs - g * L, v, NEG))
            loc[pl.ds((40 + s) * L, L)] = acc

    def process(buf, width, col0, t):
        newt = []
        for s in range(8):
            ts = t[s]
            for g in range(width // L):
                ts = _insert(ts, buf[s, pl.ds(g * L, L)])
            newt.append(ts)
            gt_probe(buf, width, col0, s)
        return tuple(newt)

    start(0, bufA, sem0)
    start(1, bufB, sem1)
    t0 = tuple(tuple(negv for _ in range(TOPK)) for _ in range(8))

    def ring(cc, t):
        c0 = cc * 2
        wait(bufA, sem0, CHUNK_COLS)
        t = process(bufA, CHUNK_COLS, colbase + c0 * CHUNK_COLS, t)

        @pl.when(c0 + 2 < NCH)
        def _():
            start(c0 + 2, bufA, sem0)

        wait(bufB, sem1, CHUNK_COLS)
        t = process(bufB, CHUNK_COLS, colbase + (c0 + 1) * CHUNK_COLS, t)

        @pl.when(c0 + 3 < NCH)
        def _():
            start(c0 + 3, bufB, sem1)

        return t

    t = lax.fori_loop(0, NCH // 2, ring, t0)

    tailc = pl.multiple_of(colbase + TAIL_COL, 128)
    pltpu.async_copy(inp.at[pl.ds(r0, 8), pl.ds(tailc, 128)], bufT, sem2)
    pltpu.make_async_copy(inp.at[pl.ds(r0, 8), pl.ds(0, 128)],
                          bufT, sem2).wait()
    t = process(bufT, 128, colbase + TAIL_COL, t)

    for s in range(8):
        for k in range(TOPK):
            loc[pl.ds((k * 8 + s) * L, L)] = t[s][k]

    @pl.when(sl == NSLAB - 1)
    def _():
        pltpu.sync_copy(inp.at[pl.ds(r0, 8), pl.ds(PART_COL, 64)], bufP)
        for s in range(8):
            ts = tuple(loc[pl.ds((k * 8 + s) * L, L)] for k in range(TOPK))
            for g in range(64 // L):
                ts = _insert(ts, bufP[s, pl.ds(g * L, L)])
            for k in range(TOPK):
                loc[pl.ds((k * 8 + s) * L, L)] = ts[k]
            gt_probe(bufP, 64, PART_COL, s)

    pltpu.sync_copy(loc, shared.at[sid])
    plsc.subcore_barrier()

    @pl.when(sid == 0)
    def _():
        pltpu.sync_copy(shared, out.at[pl.ds(core * L, L), :])


_scan_kernel = functools.partial(
    pl.kernel,
    out_type=jax.ShapeDtypeStruct((NW, 768), jnp.float32),
    mesh=plsc.VectorSubcoreMesh(core_axis_name="c", subcore_axis_name="s"),
    scratch_types=[
        pltpu.VMEM((8, CHUNK_COLS), jnp.float32),
        pltpu.VMEM((8, CHUNK_COLS), jnp.float32),
        pltpu.VMEM((8, 128), jnp.float32),
        pltpu.VMEM((8, 64), jnp.float32),
        pltpu.VMEM((2 * L,), jnp.int32),
        pltpu.VMEM((768,), jnp.float32),
        pltpu.VMEM_SHARED((16, 768), jnp.float32),
        pltpu.SemaphoreType.DMA,
        pltpu.SemaphoreType.DMA,
        pltpu.SemaphoreType.DMA,
    ],
    compiler_params=pltpu.CompilerParams(use_tc_tiling_on_sc=True),
)(_scan_body)


def _acc_body(cand_ref, gtc_ref, o_ref):
    gt = jnp.max(gtc_ref[...], axis=1)
    x = cand_ref[...]
    idxv = lax.broadcasted_iota(jnp.int32, x.shape, 1)
    for _ in range(TOPK - 1):
        m = jnp.max(x, axis=1, keepdims=True)
        cidx = jnp.where(x == m, idxv, x.shape[1])
        first = jnp.min(cidx, axis=1, keepdims=True)
        x = jnp.where(idxv == first, NEG, x)
    thr = jnp.max(x, axis=1)
    cmp = (gt[:, None] >= thr[None, :]).astype(jnp.float32)
    o_ref[...] = (jnp.sum(cmp) * (1.0 / (B * B))).reshape(1, 1)


def kernel(input, target):
    tgt = target.astype(jnp.int32)
    raw = _scan_kernel(input, tgt)
    x = raw.reshape(8, NSLAB, 6, 8, L)
    cand = x[:, :, :TOPK].transpose(0, 3, 1, 2, 4).reshape(B, NSLAB * TOPK * L)
    gtc = x[:, :, TOPK].transpose(0, 2, 1, 3).reshape(B, NSLAB * L)
    acc = pl.pallas_call(
        _acc_body,
        out_shape=jax.ShapeDtypeStruct((1, 1), jnp.float32),
    )(cand, gtc)
    return acc[0, 0]


# speedup vs baseline: 5.5809x; 1.3087x over previous
import functools

import jax
import jax.numpy as jnp
from jax import lax
from jax.experimental import pallas as pl
from jax.experimental.pallas import tpu as pltpu
from jax.experimental.pallas import tpu_sc as plsc

B = 64
N = 1000000
TOPK = 5
L = 16
NC = 2
NS = 16
NW = NC * NS
NSLAB = 4
TILES_PER_SLAB = 1953
SLAB_COLS = TILES_PER_SLAB * 128
CHUNK_COLS = 256
NCH = 976
TAIL_COL = NCH * CHUNK_COLS
PART_COL = NSLAB * SLAB_COLS

NEG = float("-inf")


def _insert(t, v):
    out = []
    r = v
    for k in range(TOPK):
        out.append(jnp.maximum(t[k], r))
        r = jnp.minimum(t[k], r)
    return tuple(out)


def _scan_body(inp, tgt, out, bufA, bufB, bufT, bufP, tgtv, loc, shared,
               sem0, sem1, sem2):
    iota = lax.iota(jnp.int32, L)
    negv = jnp.full((L,), NEG, jnp.float32)
    core = lax.axis_index("c")
    sid = lax.axis_index("s")
    wid = core * NS + sid
    tr = wid // NSLAB
    sl = wid - tr * NSLAB
    r0 = pl.multiple_of(tr * 8, 8)
    colbase = sl * SLAB_COLS

    off0 = pl.multiple_of(jnp.minimum(r0, B - L), 8)
    pltpu.sync_copy(tgt.at[pl.ds(off0, L)], tgtv.at[pl.ds(0, L)])
    pltpu.sync_copy(tgt.at[pl.ds(off0, L)], tgtv.at[pl.ds(L, L)])
    idxbase = r0 - off0
    tcols = [tgtv[pl.ds(idxbase + s, L)][0] for s in range(8)]

    for s in range(8):
        loc[pl.ds((40 + s) * L, L)] = negv

    def start(c, buf, sem):
        col = pl.multiple_of(colbase + c * CHUNK_COLS, 128)
        pltpu.async_copy(inp.at[pl.ds(r0, 8), pl.ds(col, CHUNK_COLS)],
                         buf, sem)

    def wait(buf, sem, width):
        pltpu.make_async_copy(inp.at[pl.ds(r0, 8), pl.ds(0, width)],
                              buf, sem).wait()

    def gt_probe(buf, width, col0, s):
        cs = tcols[s] - col0

        @pl.when((cs >= 0) & (cs < width))
        def _():
            acc = loc[pl.ds((40 + s) * L, L)]
            for g in range(width // L):
                v = buf[s, pl.ds(g * L, L)]
                acc = jnp.maximum(acc, jnp.where(iota == cs - g * L, v, NEG))
            loc[pl.ds((40 + s) * L, L)] = acc

    def process(buf, width, col0):
        for s in range(8):
            ts = tuple(loc[pl.ds((k * 8 + s) * L, L)] for k in range(TOPK))
            for g in range(width // L):
                ts = _insert(ts, buf[s, pl.ds(g * L, L)])
            for k in range(TOPK):
                loc[pl.ds((k * 8 + s) * L, L)] = ts[k]
            gt_probe(buf, width, col0, s)

    start(0, bufA, sem0)
    start(1, bufB, sem1)
    for s in range(8):
        for k in range(TOPK):
            loc[pl.ds((k * 8 + s) * L, L)] = negv

    def ring(cc, carry):
        c0 = cc * 2
        wait(bufA, sem0, CHUNK_COLS)
        process(bufA, CHUNK_COLS, colbase + c0 * CHUNK_COLS)

        @pl.when(c0 + 2 < NCH)
        def _():
            start(c0 + 2, bufA, sem0)

        wait(bufB, sem1, CHUNK_COLS)
        process(bufB, CHUNK_COLS, colbase + (c0 + 1) * CHUNK_COLS)

        @pl.when(c0 + 3 < NCH)
        def _():
            start(c0 + 3, bufB, sem1)

        return carry

    lax.fori_loop(0, NCH // 2, ring, 0)

    tailc = pl.multiple_of(colbase + TAIL_COL, 128)
    pltpu.async_copy(inp.at[pl.ds(r0, 8), pl.ds(tailc, 128)], bufT, sem2)
    pltpu.make_async_copy(inp.at[pl.ds(r0, 8), pl.ds(0, 128)],
                          bufT, sem2).wait()
    process(bufT, 128, colbase + TAIL_COL)

    @pl.when(sl == NSLAB - 1)
    def _():
        pltpu.sync_copy(inp.at[pl.ds(r0, 8), pl.ds(PART_COL, 64)], bufP)
        process(bufP, 64, PART_COL)

    pltpu.sync_copy(loc, shared.at[sid])
    plsc.subcore_barrier()

    @pl.when(sid == 0)
    def _():
        pltpu.sync_copy(shared, out.at[pl.ds(core * L, L), :])


_scan_kernel = functools.partial(
    pl.kernel,
    out_type=jax.ShapeDtypeStruct((NW, 768), jnp.float32),
    mesh=plsc.VectorSubcoreMesh(core_axis_name="c", subcore_axis_name="s"),
    scratch_types=[
        pltpu.VMEM((8, CHUNK_COLS), jnp.float32),
        pltpu.VMEM((8, CHUNK_COLS), jnp.float32),
        pltpu.VMEM((8, 128), jnp.float32),
        pltpu.VMEM((8, 64), jnp.float32),
        pltpu.VMEM((2 * L,), jnp.int32),
        pltpu.VMEM((768,), jnp.float32),
        pltpu.VMEM_SHARED((16, 768), jnp.float32),
        pltpu.SemaphoreType.DMA,
        pltpu.SemaphoreType.DMA,
        pltpu.SemaphoreType.DMA,
    ],
    compiler_params=pltpu.CompilerParams(use_tc_tiling_on_sc=True),
)(_scan_body)


def _acc_body(cand_ref, gtc_ref, o_ref):
    gt = jnp.max(gtc_ref[...], axis=1)
    x = cand_ref[...]
    idxv = lax.broadcasted_iota(jnp.int32, x.shape, 1)
    for _ in range(TOPK - 1):
        m = jnp.max(x, axis=1, keepdims=True)
        cidx = jnp.where(x == m, idxv, x.shape[1])
        first = jnp.min(cidx, axis=1, keepdims=True)
        x = jnp.where(idxv == first, NEG, x)
    thr = jnp.max(x, axis=1)
    cmp = (gt[:, None] >= thr[None, :]).astype(jnp.float32)
    o_ref[...] = (jnp.sum(cmp) * (1.0 / (B * B))).reshape(1, 1)


def kernel(input, target):
    tgt = target.astype(jnp.int32)
    raw = _scan_kernel(input, tgt)
    x = raw.reshape(8, NSLAB, 6, 8, L)
    cand = x[:, :, :TOPK].transpose(0, 3, 1, 2, 4).reshape(B, NSLAB * TOPK * L)
    gtc = x[:, :, TOPK].transpose(0, 2, 1, 3).reshape(B, NSLAB * L)
    acc = pl.pallas_call(
        _acc_body,
        out_shape=jax.ShapeDtypeStruct((1, 1), jnp.float32),
    )(cand, gtc)
    return acc[0, 0]


# speedup vs baseline: 20.7403x; 3.7163x over previous
import functools

import jax
import jax.numpy as jnp
from jax import lax
from jax.experimental import pallas as pl
from jax.experimental.pallas import tpu as pltpu
from jax.experimental.pallas import tpu_sc as plsc

B = 64
N = 1000000
TOPK = 5
L = 16
NC = 2
NS = 16
NW = NC * NS
NSLAB = 4
TILES_PER_SLAB = 1953
SLAB_COLS = TILES_PER_SLAB * 128
CHUNK_COLS = 2048
NCH = 122
TAIL_COL = NCH * CHUNK_COLS
PART_COL = NSLAB * SLAB_COLS

NEG = float("-inf")


def _insert(t, v):
    out = []
    r = v
    for k in range(TOPK):
        out.append(jnp.maximum(t[k], r))
        r = jnp.minimum(t[k], r)
    return tuple(out)


def _scan_body(inp, tgt, out, bufA, bufB, bufT, bufP, tgtv, loc, shared,
               sem0, sem1, sem2):
    iota = lax.iota(jnp.int32, L)
    negv = jnp.full((L,), NEG, jnp.float32)
    core = lax.axis_index("c")
    sid = lax.axis_index("s")
    wid = core * NS + sid
    tr = wid // NSLAB
    sl = wid - tr * NSLAB
    r0 = pl.multiple_of(tr * 8, 8)
    colbase = sl * SLAB_COLS

    off0 = pl.multiple_of(jnp.minimum(r0, B - L), 8)
    pltpu.sync_copy(tgt.at[pl.ds(off0, L)], tgtv.at[pl.ds(0, L)])
    pltpu.sync_copy(tgt.at[pl.ds(off0, L)], tgtv.at[pl.ds(L, L)])
    idxbase = r0 - off0
    tcols = [tgtv[pl.ds(idxbase + s, L)][0] for s in range(8)]

    for s in range(8):
        loc[pl.ds((40 + s) * L, L)] = negv

    def start(c, buf, sem):
        col = pl.multiple_of(colbase + c * CHUNK_COLS, 128)
        pltpu.async_copy(inp.at[pl.ds(r0, 8), pl.ds(col, CHUNK_COLS)],
                         buf, sem)

    def wait(buf, sem, width):
        pltpu.make_async_copy(inp.at[pl.ds(r0, 8), pl.ds(0, width)],
                              buf, sem).wait()

    def gt_probe(buf, width, col0, s):
        cs = tcols[s] - col0

        @pl.when((cs >= 0) & (cs < width))
        def _():
            def gstep(g, acc):
                v = buf[s, pl.ds(g * L, L)]
                return jnp.maximum(acc, jnp.where(iota == cs - g * L, v, NEG))

            acc = lax.fori_loop(0, width // L, gstep,
                                loc[pl.ds((40 + s) * L, L)])
            loc[pl.ds((40 + s) * L, L)] = acc

    def process(buf, width, col0):
        for s in range(8):
            ts = tuple(loc[pl.ds((k * 8 + s) * L, L)] for k in range(TOPK))
            if width >= 128:

                def tile_step(tt, ts, s=s):
                    base = tt * 128
                    for u in range(8):
                        ts = _insert(ts, buf[s, pl.ds(base + u * L, L)])
                    return ts

                ts = lax.fori_loop(0, width // 128, tile_step, ts)
            else:
                for g in range(width // L):
                    ts = _insert(ts, buf[s, pl.ds(g * L, L)])
            for k in range(TOPK):
                loc[pl.ds((k * 8 + s) * L, L)] = ts[k]
            gt_probe(buf, width, col0, s)

    start(0, bufA, sem0)
    start(1, bufB, sem1)
    for s in range(8):
        for k in range(TOPK):
            loc[pl.ds((k * 8 + s) * L, L)] = negv

    def ring(cc, carry):
        c0 = cc * 2
        wait(bufA, sem0, CHUNK_COLS)
        process(bufA, CHUNK_COLS, colbase + c0 * CHUNK_COLS)

        @pl.when(c0 + 2 < NCH)
        def _():
            start(c0 + 2, bufA, sem0)

        wait(bufB, sem1, CHUNK_COLS)
        process(bufB, CHUNK_COLS, colbase + (c0 + 1) * CHUNK_COLS)

        @pl.when(c0 + 3 < NCH)
        def _():
            start(c0 + 3, bufB, sem1)

        return carry

    lax.fori_loop(0, NCH // 2, ring, 0)

    tailc = pl.multiple_of(colbase + TAIL_COL, 128)
    pltpu.async_copy(inp.at[pl.ds(r0, 8), pl.ds(tailc, 128)], bufT, sem2)
    pltpu.make_async_copy(inp.at[pl.ds(r0, 8), pl.ds(0, 128)],
                          bufT, sem2).wait()
    process(bufT, 128, colbase + TAIL_COL)

    @pl.when(sl == NSLAB - 1)
    def _():
        pltpu.sync_copy(inp.at[pl.ds(r0, 8), pl.ds(PART_COL, 64)], bufP)
        process(bufP, 64, PART_COL)

    pltpu.sync_copy(loc, shared.at[sid])
    plsc.subcore_barrier()

    @pl.when(sid == 0)
    def _():
        pltpu.sync_copy(shared, out.at[pl.ds(core * L, L), :])


_scan_kernel = functools.partial(
    pl.kernel,
    out_type=jax.ShapeDtypeStruct((NW, 768), jnp.float32),
    mesh=plsc.VectorSubcoreMesh(core_axis_name="c", subcore_axis_name="s"),
    scratch_types=[
        pltpu.VMEM((8, CHUNK_COLS), jnp.float32),
        pltpu.VMEM((8, CHUNK_COLS), jnp.float32),
        pltpu.VMEM((8, 128), jnp.float32),
        pltpu.VMEM((8, 64), jnp.float32),
        pltpu.VMEM((2 * L,), jnp.int32),
        pltpu.VMEM((768,), jnp.float32),
        pltpu.VMEM_SHARED((16, 768), jnp.float32),
        pltpu.SemaphoreType.DMA,
        pltpu.SemaphoreType.DMA,
        pltpu.SemaphoreType.DMA,
    ],
    compiler_params=pltpu.CompilerParams(use_tc_tiling_on_sc=True),
)(_scan_body)


def _acc_body(cand_ref, gtc_ref, o_ref):
    gt = jnp.max(gtc_ref[...], axis=1)
    x = cand_ref[...]
    idxv = lax.broadcasted_iota(jnp.int32, x.shape, 1)
    for _ in range(TOPK - 1):
        m = jnp.max(x, axis=1, keepdims=True)
        cidx = jnp.where(x == m, idxv, x.shape[1])
        first = jnp.min(cidx, axis=1, keepdims=True)
        x = jnp.where(idxv == first, NEG, x)
    thr = jnp.max(x, axis=1)
    cmp = (gt[:, None] >= thr[None, :]).astype(jnp.float32)
    o_ref[...] = (jnp.sum(cmp) * (1.0 / (B * B))).reshape(1, 1)


def kernel(input, target):
    tgt = target.astype(jnp.int32)
    raw = _scan_kernel(input, tgt)
    x = raw.reshape(8, NSLAB, 6, 8, L)
    cand = x[:, :, :TOPK].transpose(0, 3, 1, 2, 4).reshape(B, NSLAB * TOPK * L)
    gtc = x[:, :, TOPK].transpose(0, 2, 1, 3).reshape(B, NSLAB * L)
    acc = pl.pallas_call(
        _acc_body,
        out_shape=jax.ShapeDtypeStruct((1, 1), jnp.float32),
    )(cand, gtc)
    return acc[0, 0]
